# Initial kernel scaffold; baseline (speedup 1.0000x reference)
import jax
import jax.numpy as jnp
from jax.experimental import pallas as pl


def kernel(entity_ids, edge_index, edge_type, emb, basis1, comp1, root1, bias1, basis2, comp2, root2, bias2):
    raise NotImplementedError("write your pallas kernel here")


# speedup vs baseline: 1.0453x; 1.0453x over previous
import functools

import jax
import jax.numpy as jnp
from jax.experimental import pallas as pl
from jax.experimental.pallas import tpu as pltpu

N = 10000
E = 160000
R = 64
D = 256
NB = 8

_ROWS = 400


def _dense_body(g_ref, w_ref, b_ref, o_ref, *, relu):
    acc = jax.lax.dot_general(
        g_ref[...], w_ref[...], (((1,), (0,)), ((), ())),
        preferred_element_type=jnp.float32)
    acc = acc + b_ref[...]
    if relu:
        acc = jnp.maximum(acc, 0.0)
    o_ref[...] = acc


def _dense_stage(gcat, wall, bias, relu):
    k = wall.shape[0]
    return pl.pallas_call(
        functools.partial(_dense_body, relu=relu),
        grid=(N // _ROWS,),
        in_specs=[
            pl.BlockSpec((_ROWS, k), lambda i: (i, 0)),
            pl.BlockSpec((k, D), lambda i: (0, 0)),
            pl.BlockSpec((1, D), lambda i: (0, 0)),
        ],
        out_specs=pl.BlockSpec((_ROWS, D), lambda i: (i, 0)),
        out_shape=jax.ShapeDtypeStruct((N, D), jnp.float32),
    )(gcat, wall, bias)


def _sparse_stage(x, src, dst, comp_e, norm):
    xs = x[src] * norm[:, None]
    parts = []
    for b in range(NB):
        parts.append(jax.ops.segment_sum(xs * comp_e[:, b:b + 1], dst, num_segments=N))
    return jnp.concatenate(parts, axis=1)


def kernel(entity_ids, edge_index, edge_type, emb, basis1, comp1, root1, bias1,
           basis2, comp2, root2, bias2):
    src = edge_index[0]
    dst = edge_index[1]
    x = emb[entity_ids]

    comb = dst * R + edge_type
    cnt = jnp.zeros((N * R,), jnp.float32).at[comb].add(1.0)
    norm = 1.0 / jnp.maximum(cnt[comb], 1.0)

    w1 = jnp.concatenate([basis1.reshape(NB * D, D), root1], axis=0)
    w2 = jnp.concatenate([basis2.reshape(NB * D, D), root2], axis=0)

    g1 = _sparse_stage(x, src, dst, comp1[edge_type], norm)
    x1 = _dense_stage(jnp.concatenate([g1, x], axis=1), w1, bias1[None, :], relu=True)
    g2 = _sparse_stage(x1, src, dst, comp2[edge_type], norm)
    x2 = _dense_stage(jnp.concatenate([g2, x1], axis=1), w2, bias2[None, :], relu=False)
    return x2


# speedup vs baseline: 2.5179x; 2.4089x over previous
import functools

import jax
import jax.numpy as jnp
from jax import lax
from jax.experimental import pallas as pl
from jax.experimental.pallas import tpu as pltpu
from jax.experimental.pallas import tpu_sc as plsc

N = 10000
E = 160000
R = 64
D = 256
NB = 8
NBD = NB * D
DL = D // 16

NC = 2
NS = 16
L = 16
NW = NC * NS
EC = E // NS
ECP = EC + 1024
OWN = 313
ACC_R = 320
HOB = NS * OWN
CNTR = HOB * R
TRASH = CNTR
CNTB = 321536
CSL = CNTB // NS
CHK = 1024

_MESH = plsc.VectorSubcoreMesh(
    core_axis_name="c", subcore_axis_name="s", num_cores=NC, num_subcores=NS)
_SC_PARAMS = pltpu.CompilerParams(needs_layout_passes=False)


def _prologue_body(src_h, dst_h, ty_h,
                   esrc_o, edl_o, ety_o, enorm_o, cnts_o,
                   sv, dv, tv, lsrc, ldl, lty, lnorm,
                   vals16, ones16, cbuf, zbuf, idxb, cnt_sh):
    c = lax.axis_index("c")
    s = lax.axis_index("s")
    pltpu.sync_copy(src_h.at[pl.ds(s * EC, EC)], sv)
    pltpu.sync_copy(dst_h.at[pl.ds(s * EC, EC)], dv)
    pltpu.sync_copy(ty_h.at[pl.ds(s * EC, EC)], tv)
    ones16[...] = jnp.ones((L,), jnp.float32)

    def zero_body(k, _):
        zbuf[pl.ds(k * L, L)] = jnp.zeros((L,), jnp.float32)
        return 0
    lax.fori_loop(0, 2048 // L, zero_body, 0)
    for k in range(9):
        pltpu.sync_copy(zbuf, cnt_sh.at[pl.ds(s * CSL + k * 2048, 2048)])
    pltpu.sync_copy(zbuf.at[pl.ds(0, CSL - 9 * 2048)],
                    cnt_sh.at[pl.ds(s * CSL + 9 * 2048, CSL - 9 * 2048)])
    plsc.subcore_barrier()

    lo = c * HOB

    def count_body(b, _):
        base = b * L
        dstv = dv[pl.ds(base, L)]
        tyv = tv[pl.ds(base, L)]
        dstl = dstv - lo
        inh = (dstv >= lo) & (dstv < lo + HOB)
        comb = dstl * R + tyv
        idxb[...] = jnp.where(inh, comb, TRASH)
        pltpu.sync_copy(
            ones16,
            cnt_sh.at[plsc.Indices(idxb.at[...], ignored_value=TRASH)],
            add=True)
        return 0
    lax.fori_loop(0, EC // L, count_body, 0)
    plsc.subcore_barrier()

    def bucket_body(k, _):
        qlo = lo + k * OWN

        def norm_body(b, off):
            base = b * L
            srcv = sv[pl.ds(base, L)]
            dstv = dv[pl.ds(base, L)]
            tyv = tv[pl.ds(base, L)]
            inq = (dstv >= qlo) & (dstv < qlo + OWN)
            comb = (dstv - lo) * R + tyv
            idxb[...] = jnp.where(inq, comb, TRASH)
            pltpu.sync_copy(cnt_sh.at[idxb.at[...]], vals16)
            cntv = vals16[...]
            normv = 1.0 / jnp.maximum(cntv, 1.0)
            mi = jnp.where(inq, 1, 0)
            fullm = lax.iota(jnp.int32, L) < L
            cbuf[...] = plsc.cumsum(mi, mask=fullm)
            cumv = cbuf[...]
            tgt = cumv - 1 + off
            dlv = dstv - qlo
            plsc.store_scatter(lsrc, [tgt], srcv, mask=inq)
            plsc.store_scatter(ldl, [tgt], dlv, mask=inq)
            plsc.store_scatter(lty, [tgt], tyv, mask=inq)
            plsc.store_scatter(lnorm, [tgt], normv, mask=inq)
            return off + cumv[L - 1]
        total = lax.fori_loop(0, EC // L, norm_body, jnp.int32(0))

        wk = c * NS + k
        slot = (s * NW + wk) * ECP
        pltpu.sync_copy(lsrc, esrc_o.at[pl.ds(slot, EC)])
        pltpu.sync_copy(ldl, edl_o.at[pl.ds(slot, EC)])
        pltpu.sync_copy(lty, ety_o.at[pl.ds(slot, EC)])
        pltpu.sync_copy(lnorm, enorm_o.at[pl.ds(slot, EC)])
        cbuf[...] = jnp.where(lax.iota(jnp.int32, L) == 0, total, 0)
        pltpu.sync_copy(cbuf, cnts_o.at[pl.ds((s * NW + wk) * L, L)])
        return 0
    lax.fori_loop(0, NS, bucket_body, 0)


def _make_prologue():
    return functools.partial(
        pl.kernel,
        out_type=[
            jax.ShapeDtypeStruct((NS * NW * ECP,), jnp.int32),
            jax.ShapeDtypeStruct((NS * NW * ECP,), jnp.int32),
            jax.ShapeDtypeStruct((NS * NW * ECP,), jnp.int32),
            jax.ShapeDtypeStruct((NS * NW * ECP,), jnp.float32),
            jax.ShapeDtypeStruct((NS * NW * L,), jnp.int32),
        ],
        mesh=_MESH,
        scratch_types=[
            pltpu.VMEM((EC,), jnp.int32),
            pltpu.VMEM((EC,), jnp.int32),
            pltpu.VMEM((EC,), jnp.int32),
            pltpu.VMEM((EC,), jnp.int32),
            pltpu.VMEM((EC,), jnp.int32),
            pltpu.VMEM((EC,), jnp.int32),
            pltpu.VMEM((EC,), jnp.float32),
            pltpu.VMEM((L,), jnp.float32),
            pltpu.VMEM((L,), jnp.float32),
            pltpu.VMEM((L,), jnp.int32),
            pltpu.VMEM((2048,), jnp.float32),
            pltpu.VMEM((L,), jnp.int32),
            pltpu.VMEM_SHARED((CNTB,), jnp.float32),
        ],
        compiler_params=_SC_PARAMS,
        name="rgcn_prologue",
    )(_prologue_body)


def _aggregate_body(esrc_h, edl_h, ety_h, enorm_h, cnts_h, comp_h, y_h,
                    apad_o,
                    sbuf, dbuf, tbuf, nbuf, compv, cbuf, yb, cfbuf, dlbuf,
                    acc, sem):
    c = lax.axis_index("c")
    s = lax.axis_index("s")
    w = c * NS + s
    pltpu.sync_copy(comp_h, compv)

    def zz(k, _):
        it = lax.iota(jnp.int32, L)
        row = k // DL
        col = (k % DL) * L
        plsc.store_scatter(acc, [jnp.full((L,), 0, jnp.int32) + row, col + it],
                           jnp.zeros((L,), jnp.float32))
        return 0
    lax.fori_loop(0, ACC_R * DL, zz, 0)

    def src_chunk(s1, _):
        slot = (s1 * NW + w) * ECP
        pltpu.sync_copy(cnts_h.at[pl.ds((s1 * NW + w) * L, L)], cbuf)
        count = cbuf[...][0]
        nch = (count + (CHK - 1)) // CHK

        def chunk_body(ch, _):
            cbase = ch * CHK
            ck = jnp.minimum(CHK, count - cbase)
            pltpu.sync_copy(esrc_h.at[pl.ds(slot + cbase, CHK)], sbuf)
            pltpu.sync_copy(edl_h.at[pl.ds(slot + cbase, CHK)], dbuf)
            pltpu.sync_copy(ety_h.at[pl.ds(slot + cbase, CHK)], tbuf)
            pltpu.sync_copy(enorm_h.at[pl.ds(slot + cbase, CHK)], nbuf)
            nblk = (ck + (L - 1)) // L

            def block_body(bk, _):
                base = bk * L
                it = lax.iota(jnp.int32, L)
                valid = (base + it) < ck
                srcv = jnp.where(valid, sbuf[pl.ds(base, L)], 0)
                pltpu.async_copy(y_h.at[srcv], yb, sem).wait()
                tyv = jnp.where(valid, tbuf[pl.ds(base, L)], 0)
                normv = jnp.where(valid, nbuf[pl.ds(base, L)], 0.0)
                for bb in range(NB):
                    cfbuf[pl.ds(bb * L, L)] = normv * plsc.load_gather(
                        compv, [tyv, jnp.full((L,), bb, jnp.int32)])
                dlbuf[...] = dbuf[pl.ds(base, L)]
                nv = jnp.minimum(L, ck - base)

                def edge_body(e, _):
                    it2 = lax.iota(jnp.int32, L)
                    ef = jnp.full((L,), 0, jnp.int32) + e
                    dls = plsc.load_gather(dlbuf, [ef])
                    cfs = [plsc.load_gather(
                        cfbuf, [jnp.full((L,), bb * L, jnp.int32) + ef])
                        for bb in range(NB)]
                    for cc in range(DL):
                        a16 = cfs[0] * yb[e, pl.ds(cc * L, L)]
                        for bb in range(1, NB):
                            a16 = a16 + cfs[bb] * yb[e,
                                                     pl.ds(bb * D + cc * L, L)]
                        plsc.addupdate_scatter(
                            acc,
                            [dls, jnp.full((L,), cc * L, jnp.int32) + it2],
                            a16)
                    return 0
                lax.fori_loop(0, nv, edge_body, 0)
                return 0
            lax.fori_loop(0, nblk, block_body, 0)
            return 0
        lax.fori_loop(0, nch, chunk_body, 0)
        return 0
    lax.fori_loop(0, NS, src_chunk, 0)

    pltpu.sync_copy(acc, apad_o.at[pl.ds(w * ACC_R, ACC_R)])


def _make_aggregate():
    return functools.partial(
        pl.kernel,
        out_type=jax.ShapeDtypeStruct((NW * ACC_R, D), jnp.float32),
        mesh=_MESH,
        scratch_types=[
            pltpu.VMEM((CHK,), jnp.int32),
            pltpu.VMEM((CHK,), jnp.int32),
            pltpu.VMEM((CHK,), jnp.int32),
            pltpu.VMEM((CHK,), jnp.float32),
            pltpu.VMEM((R, NB), jnp.float32),
            pltpu.VMEM((L,), jnp.int32),
            pltpu.VMEM((L, NBD), jnp.float32),
            pltpu.VMEM((NB * L,), jnp.float32),
            pltpu.VMEM((L,), jnp.int32),
            pltpu.VMEM((ACC_R, D), jnp.float32),
            pltpu.SemaphoreType.DMA,
        ],
        compiler_params=_SC_PARAMS,
        name="rgcn_aggregate",
    )(_aggregate_body)


_ROWS = 400


def _mm1_body(x_ref, w_ref, yb_ref, yr_ref):
    acc = lax.dot_general(x_ref[...], w_ref[...], (((1,), (0,)), ((), ())),
                          preferred_element_type=jnp.float32)
    yb_ref[...] = acc[:, :NBD]
    yr_ref[...] = acc[:, NBD:]


def _mm1(x, w):
    return pl.pallas_call(
        _mm1_body,
        grid=(N // _ROWS,),
        in_specs=[
            pl.BlockSpec((_ROWS, D), lambda i: (i, 0)),
            pl.BlockSpec((D, NBD + D), lambda i: (0, 0)),
        ],
        out_specs=[
            pl.BlockSpec((_ROWS, NBD), lambda i: (i, 0)),
            pl.BlockSpec((_ROWS, D), lambda i: (i, 0)),
        ],
        out_shape=[
            jax.ShapeDtypeStruct((N, NBD), jnp.float32),
            jax.ShapeDtypeStruct((N, D), jnp.float32),
        ],
    )(x, w)


def _mm2_body(a_ref, yr_ref, b_ref, w_ref, yb_ref, yr2_ref):
    x1 = jnp.maximum(a_ref[...] + yr_ref[...] + b_ref[...], 0.0)
    acc = lax.dot_general(x1, w_ref[...], (((1,), (0,)), ((), ())),
                          preferred_element_type=jnp.float32)
    yb_ref[...] = acc[:, :NBD]
    yr2_ref[...] = acc[:, NBD:]


def _mm2(a1, yr1, bias1, w2):
    return pl.pallas_call(
        _mm2_body,
        grid=(N // _ROWS,),
        in_specs=[
            pl.BlockSpec((_ROWS, D), lambda i: (i, 0)),
            pl.BlockSpec((_ROWS, D), lambda i: (i, 0)),
            pl.BlockSpec((1, D), lambda i: (0, 0)),
            pl.BlockSpec((D, NBD + D), lambda i: (0, 0)),
        ],
        out_specs=[
            pl.BlockSpec((_ROWS, NBD), lambda i: (i, 0)),
            pl.BlockSpec((_ROWS, D), lambda i: (i, 0)),
        ],
        out_shape=[
            jax.ShapeDtypeStruct((N, NBD), jnp.float32),
            jax.ShapeDtypeStruct((N, D), jnp.float32),
        ],
    )(a1, yr1, bias1, w2)


def _fin_body(a_ref, yr_ref, b_ref, o_ref):
    o_ref[...] = a_ref[...] + yr_ref[...] + b_ref[...]


def _fin(a2, yr2, bias2):
    return pl.pallas_call(
        _fin_body,
        grid=(N // _ROWS,),
        in_specs=[
            pl.BlockSpec((_ROWS, D), lambda i: (i, 0)),
            pl.BlockSpec((_ROWS, D), lambda i: (i, 0)),
            pl.BlockSpec((1, D), lambda i: (0, 0)),
        ],
        out_specs=pl.BlockSpec((_ROWS, D), lambda i: (i, 0)),
        out_shape=jax.ShapeDtypeStruct((N, D), jnp.float32),
    )(a2, yr2, bias2)


def kernel(entity_ids, edge_index, edge_type, emb, basis1, comp1, root1, bias1,
           basis2, comp2, root2, bias2):
    src = edge_index[0]
    dst = edge_index[1]
    x = emb[entity_ids]

    esrc, edl, ety, enorm, cnts = _make_prologue()(src, dst, edge_type)

    w1 = jnp.concatenate(
        [basis1.transpose(1, 0, 2).reshape(D, NBD), root1], axis=1)
    w2 = jnp.concatenate(
        [basis2.transpose(1, 0, 2).reshape(D, NBD), root2], axis=1)

    def _unpad(apad):
        return apad.reshape(NW, ACC_R, D)[:, :OWN].reshape(NW * OWN, D)[:N]

    yb1, yr1 = _mm1(x, w1)
    a1 = _unpad(_make_aggregate()(esrc, edl, ety, enorm, cnts, comp1, yb1))
    yb2, yr2 = _mm2(a1, yr1, bias1[None, :], w2)
    a2 = _unpad(_make_aggregate()(esrc, edl, ety, enorm, cnts, comp2, yb2))
    return _fin(a2, yr2, bias2[None, :])


# speedup vs baseline: 4.7432x; 1.8838x over previous
import functools

import jax
import jax.numpy as jnp
from jax import lax
from jax.experimental import pallas as pl
from jax.experimental.pallas import tpu as pltpu
from jax.experimental.pallas import tpu_sc as plsc

N = 10000
E = 160000
R = 64
D = 256
NB = 8
NBD = NB * D
DL = D // 16

NC = 2
NS = 16
L = 16
NW = NC * NS
EC = E // NS
ECP = EC + 1024
OWN = 313
ACC_R = 320
HOB = NS * OWN
CNTR = HOB * R
TRASH = CNTR
CNTB = 321536
CSL = CNTB // NS
CHK = 1024

_MESH = plsc.VectorSubcoreMesh(
    core_axis_name="c", subcore_axis_name="s", num_cores=NC, num_subcores=NS)
_SC_PARAMS = pltpu.CompilerParams(needs_layout_passes=False)


def _prologue_body(src_h, dst_h, ty_h,
                   esrc_o, edl_o, ety_o, enorm_o, cnts_o,
                   sv, dv, tv, lsrc, ldl, lty, lnorm, nrmb,
                   vals16, ones16, cbuf, zbuf, idxb, cnt_sh):
    c = lax.axis_index("c")
    s = lax.axis_index("s")
    pltpu.sync_copy(src_h.at[pl.ds(s * EC, EC)], sv)
    pltpu.sync_copy(dst_h.at[pl.ds(s * EC, EC)], dv)
    pltpu.sync_copy(ty_h.at[pl.ds(s * EC, EC)], tv)
    ones16[...] = jnp.ones((L,), jnp.float32)

    def zero_body(k, _):
        zbuf[pl.ds(k * L, L)] = jnp.zeros((L,), jnp.float32)
        return 0
    lax.fori_loop(0, 2048 // L, zero_body, 0)
    for k in range(9):
        pltpu.sync_copy(zbuf, cnt_sh.at[pl.ds(s * CSL + k * 2048, 2048)])
    pltpu.sync_copy(zbuf.at[pl.ds(0, CSL - 9 * 2048)],
                    cnt_sh.at[pl.ds(s * CSL + 9 * 2048, CSL - 9 * 2048)])
    plsc.subcore_barrier()

    lo = c * HOB

    def count_body(b, _):
        base = b * L
        dstv = dv[pl.ds(base, L)]
        tyv = tv[pl.ds(base, L)]
        dstl = dstv - lo
        inh = (dstv >= lo) & (dstv < lo + HOB)
        comb = dstl * R + tyv
        idxb[...] = jnp.where(inh, comb, TRASH)
        pltpu.sync_copy(
            ones16,
            cnt_sh.at[plsc.Indices(idxb.at[...], ignored_value=TRASH)],
            add=True)
        return 0
    lax.fori_loop(0, EC // L, count_body, 0)
    plsc.subcore_barrier()

    def prenorm_body(b, _):
        base = b * L
        dstv = dv[pl.ds(base, L)]
        tyv = tv[pl.ds(base, L)]
        inh = (dstv >= lo) & (dstv < lo + HOB)
        comb = (dstv - lo) * R + tyv
        idxb[...] = jnp.where(inh, comb, TRASH)
        pltpu.sync_copy(cnt_sh.at[idxb.at[...]], vals16)
        cntv = vals16[...]
        nrmb[pl.ds(base, L)] = 1.0 / jnp.maximum(cntv, 1.0)
        return 0
    lax.fori_loop(0, EC // L, prenorm_body, 0)

    def bucket_body(k, _):
        qlo = lo + k * OWN

        def norm_body(b, off):
            base = b * L
            srcv = sv[pl.ds(base, L)]
            dstv = dv[pl.ds(base, L)]
            tyv = tv[pl.ds(base, L)]
            normv = nrmb[pl.ds(base, L)]
            inq = (dstv >= qlo) & (dstv < qlo + OWN)
            mi = jnp.where(inq, 1, 0)
            fullm = lax.iota(jnp.int32, L) < L
            cbuf[...] = plsc.cumsum(mi, mask=fullm)
            cumv = cbuf[...]
            tgt = cumv - 1 + off
            dlv = dstv - qlo
            plsc.store_scatter(lsrc, [tgt], srcv, mask=inq)
            plsc.store_scatter(ldl, [tgt], dlv, mask=inq)
            plsc.store_scatter(lty, [tgt], tyv, mask=inq)
            plsc.store_scatter(lnorm, [tgt], normv, mask=inq)
            return off + cumv[L - 1]
        total = lax.fori_loop(0, EC // L, norm_body, jnp.int32(0))

        wk = c * NS + k
        slot = (s * NW + wk) * ECP
        pltpu.sync_copy(lsrc, esrc_o.at[pl.ds(slot, EC)])
        pltpu.sync_copy(ldl, edl_o.at[pl.ds(slot, EC)])
        pltpu.sync_copy(lty, ety_o.at[pl.ds(slot, EC)])
        pltpu.sync_copy(lnorm, enorm_o.at[pl.ds(slot, EC)])
        cbuf[...] = jnp.where(lax.iota(jnp.int32, L) == 0, total, 0)
        pltpu.sync_copy(cbuf, cnts_o.at[pl.ds((s * NW + wk) * L, L)])
        return 0
    lax.fori_loop(0, NS, bucket_body, 0)


def _make_prologue():
    return functools.partial(
        pl.kernel,
        out_type=[
            jax.ShapeDtypeStruct((NS * NW * ECP,), jnp.int32),
            jax.ShapeDtypeStruct((NS * NW * ECP,), jnp.int32),
            jax.ShapeDtypeStruct((NS * NW * ECP,), jnp.int32),
            jax.ShapeDtypeStruct((NS * NW * ECP,), jnp.float32),
            jax.ShapeDtypeStruct((NS * NW * L,), jnp.int32),
        ],
        mesh=_MESH,
        scratch_types=[
            pltpu.VMEM((EC,), jnp.int32),
            pltpu.VMEM((EC,), jnp.int32),
            pltpu.VMEM((EC,), jnp.int32),
            pltpu.VMEM((EC,), jnp.int32),
            pltpu.VMEM((EC,), jnp.int32),
            pltpu.VMEM((EC,), jnp.int32),
            pltpu.VMEM((EC,), jnp.float32),
            pltpu.VMEM((EC,), jnp.float32),
            pltpu.VMEM((L,), jnp.float32),
            pltpu.VMEM((L,), jnp.float32),
            pltpu.VMEM((L,), jnp.int32),
            pltpu.VMEM((2048,), jnp.float32),
            pltpu.VMEM((L,), jnp.int32),
            pltpu.VMEM_SHARED((CNTB,), jnp.float32),
        ],
        compiler_params=_SC_PARAMS,
        name="rgcn_prologue",
    )(_prologue_body)


HB = 8


def _edge_compute(ybuf, e, lane, cfbuf, dlbuf, acc):
    it2 = lax.iota(jnp.int32, L)
    ef = jnp.full((L,), 0, jnp.int32) + lane
    dls = plsc.load_gather(dlbuf, [ef])
    cfs = [plsc.load_gather(cfbuf, [jnp.full((L,), bb * L, jnp.int32) + ef])
           for bb in range(NB)]
    for cc in range(DL):
        ys = [ybuf[e, pl.ds(bb * D + cc * L, L)] for bb in range(NB)]
        t0 = cfs[0] * ys[0] + cfs[1] * ys[1]
        t1 = cfs[2] * ys[2] + cfs[3] * ys[3]
        t2 = cfs[4] * ys[4] + cfs[5] * ys[5]
        t3 = cfs[6] * ys[6] + cfs[7] * ys[7]
        a16 = (t0 + t1) + (t2 + t3)
        plsc.addupdate_scatter(
            acc, [dls, jnp.full((L,), cc * L, jnp.int32) + it2], a16)


def _aggregate_body(esrc_h, edl_h, ety_h, enorm_h, cnts_h, comp_h, y_h,
                    apad_o,
                    sbuf, dbuf, tbuf, nbuf, compv, cbuf, ybA, ybB, cfbuf,
                    dlbuf, idxb, acc, semA, semB):
    c = lax.axis_index("c")
    s = lax.axis_index("s")
    w = c * NS + s
    pltpu.sync_copy(comp_h, compv)

    def zz(k, _):
        it = lax.iota(jnp.int32, L)
        row = k // DL
        col = (k % DL) * L
        plsc.store_scatter(acc, [jnp.full((L,), 0, jnp.int32) + row, col + it],
                           jnp.zeros((L,), jnp.float32))
        return 0
    lax.fori_loop(0, ACC_R * DL, zz, 0)

    def src_chunk(s1, _):
        slot = (s1 * NW + w) * ECP
        pltpu.sync_copy(cnts_h.at[pl.ds((s1 * NW + w) * L, L)], cbuf)
        count = cbuf[...][0]
        nch = (count + (CHK - 1)) // CHK

        def chunk_body(ch, _):
            cbase = ch * CHK
            ck = jnp.minimum(CHK, count - cbase)
            pltpu.sync_copy(esrc_h.at[pl.ds(slot + cbase, CHK)], sbuf)
            pltpu.sync_copy(edl_h.at[pl.ds(slot + cbase, CHK)], dbuf)
            pltpu.sync_copy(ety_h.at[pl.ds(slot + cbase, CHK)], tbuf)
            pltpu.sync_copy(enorm_h.at[pl.ds(slot + cbase, CHK)], nbuf)
            nblk = (ck + (L - 1)) // L

            it0 = lax.iota(jnp.int32, L)
            idxb[...] = jnp.where(it0 < ck, sbuf[pl.ds(0, L)], 0)
            pltpu.async_copy(y_h.at[idxb.at[pl.ds(0, HB)]], ybA, semA)

            def block_body(bk, _):
                base = bk * L
                it = lax.iota(jnp.int32, L)
                valid = (base + it) < ck
                tyv = jnp.where(valid, tbuf[pl.ds(base, L)], 0)
                normv = jnp.where(valid, nbuf[pl.ds(base, L)], 0.0)
                for bb in range(NB):
                    cfbuf[pl.ds(bb * L, L)] = normv * plsc.load_gather(
                        compv, [tyv, jnp.full((L,), bb, jnp.int32)])
                dlbuf[...] = dbuf[pl.ds(base, L)]
                nv = jnp.minimum(L, ck - base)

                pltpu.make_async_copy(
                    y_h.at[pl.ds(0, HB)], ybA, semA).wait()
                pltpu.async_copy(y_h.at[idxb.at[pl.ds(HB, HB)]], ybB, semB)

                nva = jnp.minimum(nv, HB)

                def edge_a(e, _):
                    _edge_compute(ybA, e, e, cfbuf, dlbuf, acc)
                    return 0
                lax.fori_loop(0, nva, edge_a, 0)

                pltpu.make_async_copy(
                    y_h.at[pl.ds(0, HB)], ybB, semB).wait()

                @pl.when(bk + 1 < nblk)
                def _():
                    base2 = base + L
                    it3 = lax.iota(jnp.int32, L)
                    v2 = (base2 + it3) < ck
                    idxb[...] = jnp.where(v2, sbuf[pl.ds(base2, L)], 0)
                    pltpu.async_copy(y_h.at[idxb.at[pl.ds(0, HB)]], ybA, semA)

                nvb = jnp.maximum(nv - HB, 0)

                def edge_b(e, _):
                    _edge_compute(ybB, e, HB + e, cfbuf, dlbuf, acc)
                    return 0
                lax.fori_loop(0, nvb, edge_b, 0)
                return 0
            lax.fori_loop(0, nblk, block_body, 0)
            return 0
        lax.fori_loop(0, nch, chunk_body, 0)
        return 0
    lax.fori_loop(0, NS, src_chunk, 0)

    pltpu.sync_copy(acc, apad_o.at[pl.ds(w * ACC_R, ACC_R)])


def _make_aggregate():
    return functools.partial(
        pl.kernel,
        out_type=jax.ShapeDtypeStruct((NW * ACC_R, D), jnp.float32),
        mesh=_MESH,
        scratch_types=[
            pltpu.VMEM((CHK,), jnp.int32),
            pltpu.VMEM((CHK,), jnp.int32),
            pltpu.VMEM((CHK,), jnp.int32),
            pltpu.VMEM((CHK,), jnp.float32),
            pltpu.VMEM((R, NB), jnp.float32),
            pltpu.VMEM((L,), jnp.int32),
            pltpu.VMEM((HB, NBD), jnp.float32),
            pltpu.VMEM((HB, NBD), jnp.float32),
            pltpu.VMEM((NB * L,), jnp.float32),
            pltpu.VMEM((L,), jnp.int32),
            pltpu.VMEM((L,), jnp.int32),
            pltpu.VMEM((ACC_R, D), jnp.float32),
            pltpu.SemaphoreType.DMA,
            pltpu.SemaphoreType.DMA,
        ],
        compiler_params=_SC_PARAMS,
        name="rgcn_aggregate",
    )(_aggregate_body)


_ROWS = 400


def _mm1_body(x_ref, w_ref, yb_ref, yr_ref):
    acc = lax.dot_general(x_ref[...], w_ref[...], (((1,), (0,)), ((), ())),
                          preferred_element_type=jnp.float32)
    yb_ref[...] = acc[:, :NBD]
    yr_ref[...] = acc[:, NBD:]


def _mm1(x, w):
    return pl.pallas_call(
        _mm1_body,
        grid=(N // _ROWS,),
        in_specs=[
            pl.BlockSpec((_ROWS, D), lambda i: (i, 0)),
            pl.BlockSpec((D, NBD + D), lambda i: (0, 0)),
        ],
        out_specs=[
            pl.BlockSpec((_ROWS, NBD), lambda i: (i, 0)),
            pl.BlockSpec((_ROWS, D), lambda i: (i, 0)),
        ],
        out_shape=[
            jax.ShapeDtypeStruct((N, NBD), jnp.float32),
            jax.ShapeDtypeStruct((N, D), jnp.float32),
        ],
    )(x, w)


def _mm2_body(a_ref, yr_ref, b_ref, w_ref, yb_ref, yr2_ref):
    x1 = jnp.maximum(a_ref[...] + yr_ref[...] + b_ref[...], 0.0)
    acc = lax.dot_general(x1, w_ref[...], (((1,), (0,)), ((), ())),
                          preferred_element_type=jnp.float32)
    yb_ref[...] = acc[:, :NBD]
    yr2_ref[...] = acc[:, NBD:]


def _mm2(a1, yr1, bias1, w2):
    return pl.pallas_call(
        _mm2_body,
        grid=(N // _ROWS,),
        in_specs=[
            pl.BlockSpec((_ROWS, D), lambda i: (i, 0)),
            pl.BlockSpec((_ROWS, D), lambda i: (i, 0)),
            pl.BlockSpec((1, D), lambda i: (0, 0)),
            pl.BlockSpec((D, NBD + D), lambda i: (0, 0)),
        ],
        out_specs=[
            pl.BlockSpec((_ROWS, NBD), lambda i: (i, 0)),
            pl.BlockSpec((_ROWS, D), lambda i: (i, 0)),
        ],
        out_shape=[
            jax.ShapeDtypeStruct((N, NBD), jnp.float32),
            jax.ShapeDtypeStruct((N, D), jnp.float32),
        ],
    )(a1, yr1, bias1, w2)


def _fin_body(a_ref, yr_ref, b_ref, o_ref):
    o_ref[...] = a_ref[...] + yr_ref[...] + b_ref[...]


def _fin(a2, yr2, bias2):
    return pl.pallas_call(
        _fin_body,
        grid=(N // _ROWS,),
        in_specs=[
            pl.BlockSpec((_ROWS, D), lambda i: (i, 0)),
            pl.BlockSpec((_ROWS, D), lambda i: (i, 0)),
            pl.BlockSpec((1, D), lambda i: (0, 0)),
        ],
        out_specs=pl.BlockSpec((_ROWS, D), lambda i: (i, 0)),
        out_shape=jax.ShapeDtypeStruct((N, D), jnp.float32),
    )(a2, yr2, bias2)


def kernel(entity_ids, edge_index, edge_type, emb, basis1, comp1, root1, bias1,
           basis2, comp2, root2, bias2):
    src = edge_index[0]
    dst = edge_index[1]
    x = emb[entity_ids]

    esrc, edl, ety, enorm, cnts = _make_prologue()(src, dst, edge_type)

    w1 = jnp.concatenate(
        [basis1.transpose(1, 0, 2).reshape(D, NBD), root1], axis=1)
    w2 = jnp.concatenate(
        [basis2.transpose(1, 0, 2).reshape(D, NBD), root2], axis=1)

    def _unpad(apad):
        return apad.reshape(NW, ACC_R, D)[:, :OWN].reshape(NW * OWN, D)[:N]

    yb1, yr1 = _mm1(x, w1)
    a1 = _unpad(_make_aggregate()(esrc, edl, ety, enorm, cnts, comp1, yb1))
    yb2, yr2 = _mm2(a1, yr1, bias1[None, :], w2)
    a2 = _unpad(_make_aggregate()(esrc, edl, ety, enorm, cnts, comp2, yb2))
    return _fin(a2, yr2, bias2[None, :])


# speedup vs baseline: 5.4204x; 1.1428x over previous
import functools

import jax
import jax.numpy as jnp
from jax import lax
from jax.experimental import pallas as pl
from jax.experimental.pallas import tpu as pltpu
from jax.experimental.pallas import tpu_sc as plsc

N = 10000
E = 160000
R = 64
D = 256
NB = 8
NBD = NB * D
DL = D // 16

NC = 2
NS = 16
L = 16
NW = NC * NS
EC = E // NS
ECP = EC + 1024
OWN = 313
ACC_R = 320
HOB = NS * OWN
CNTR = HOB * R
TRASH = CNTR
CNTB = 321536
CSL = CNTB // NS
CHK = 1024

_MESH = plsc.VectorSubcoreMesh(
    core_axis_name="c", subcore_axis_name="s", num_cores=NC, num_subcores=NS)
_SC_PARAMS = pltpu.CompilerParams(needs_layout_passes=False)


def _prologue_body(src_h, dst_h, ty_h,
                   esrc_o, edl_o, ety_o, enorm_o, cnts_o,
                   sv, dv, tv, lsrc, ldl, lty, lnorm, nrmb,
                   vals16, ones16, cbuf, zbuf, idxb, cnt_sh):
    c = lax.axis_index("c")
    s = lax.axis_index("s")
    pltpu.sync_copy(src_h.at[pl.ds(s * EC, EC)], sv)
    pltpu.sync_copy(dst_h.at[pl.ds(s * EC, EC)], dv)
    pltpu.sync_copy(ty_h.at[pl.ds(s * EC, EC)], tv)
    ones16[...] = jnp.ones((L,), jnp.float32)

    def zero_body(k, _):
        zbuf[pl.ds(k * L, L)] = jnp.zeros((L,), jnp.float32)
        return 0
    lax.fori_loop(0, 2048 // L, zero_body, 0)
    for k in range(9):
        pltpu.sync_copy(zbuf, cnt_sh.at[pl.ds(s * CSL + k * 2048, 2048)])
    pltpu.sync_copy(zbuf.at[pl.ds(0, CSL - 9 * 2048)],
                    cnt_sh.at[pl.ds(s * CSL + 9 * 2048, CSL - 9 * 2048)])
    plsc.subcore_barrier()

    lo = c * HOB

    def count_body(b, _):
        base = b * L
        dstv = dv[pl.ds(base, L)]
        tyv = tv[pl.ds(base, L)]
        dstl = dstv - lo
        inh = (dstv >= lo) & (dstv < lo + HOB)
        comb = dstl * R + tyv
        idxb[...] = jnp.where(inh, comb, TRASH)
        pltpu.sync_copy(
            ones16,
            cnt_sh.at[plsc.Indices(idxb.at[...], ignored_value=TRASH)],
            add=True)
        return 0
    lax.fori_loop(0, EC // L, count_body, 0)
    plsc.subcore_barrier()

    def prenorm_body(b, _):
        base = b * L
        dstv = dv[pl.ds(base, L)]
        tyv = tv[pl.ds(base, L)]
        inh = (dstv >= lo) & (dstv < lo + HOB)
        comb = (dstv - lo) * R + tyv
        idxb[...] = jnp.where(inh, comb, TRASH)
        pltpu.sync_copy(cnt_sh.at[idxb.at[...]], vals16)
        cntv = vals16[...]
        nrmb[pl.ds(base, L)] = 1.0 / jnp.maximum(cntv, 1.0)
        return 0
    lax.fori_loop(0, EC // L, prenorm_body, 0)

    def bucket_body(k, _):
        qlo = lo + k * OWN

        def norm_body(b, off):
            base = b * L
            srcv = sv[pl.ds(base, L)]
            dstv = dv[pl.ds(base, L)]
            tyv = tv[pl.ds(base, L)]
            normv = nrmb[pl.ds(base, L)]
            inq = (dstv >= qlo) & (dstv < qlo + OWN)
            mi = jnp.where(inq, 1, 0)
            fullm = lax.iota(jnp.int32, L) < L
            cbuf[...] = plsc.cumsum(mi, mask=fullm)
            cumv = cbuf[...]
            tgt = cumv - 1 + off
            dlv = dstv - qlo
            plsc.store_scatter(lsrc, [tgt], srcv, mask=inq)
            plsc.store_scatter(ldl, [tgt], dlv, mask=inq)
            plsc.store_scatter(lty, [tgt], tyv, mask=inq)
            plsc.store_scatter(lnorm, [tgt], normv, mask=inq)
            return off + cumv[L - 1]
        total = lax.fori_loop(0, EC // L, norm_body, jnp.int32(0))

        wk = c * NS + k
        slot = (s * NW + wk) * ECP
        pltpu.sync_copy(lsrc, esrc_o.at[pl.ds(slot, EC)])
        pltpu.sync_copy(ldl, edl_o.at[pl.ds(slot, EC)])
        pltpu.sync_copy(lty, ety_o.at[pl.ds(slot, EC)])
        pltpu.sync_copy(lnorm, enorm_o.at[pl.ds(slot, EC)])
        cbuf[...] = jnp.where(lax.iota(jnp.int32, L) == 0, total, 0)
        pltpu.sync_copy(cbuf, cnts_o.at[pl.ds((s * NW + wk) * L, L)])
        return 0
    lax.fori_loop(0, NS, bucket_body, 0)


def _make_prologue():
    return functools.partial(
        pl.kernel,
        out_type=[
            jax.ShapeDtypeStruct((NS * NW * ECP,), jnp.int32),
            jax.ShapeDtypeStruct((NS * NW * ECP,), jnp.int32),
            jax.ShapeDtypeStruct((NS * NW * ECP,), jnp.int32),
            jax.ShapeDtypeStruct((NS * NW * ECP,), jnp.float32),
            jax.ShapeDtypeStruct((NS * NW * L,), jnp.int32),
        ],
        mesh=_MESH,
        scratch_types=[
            pltpu.VMEM((EC,), jnp.int32),
            pltpu.VMEM((EC,), jnp.int32),
            pltpu.VMEM((EC,), jnp.int32),
            pltpu.VMEM((EC,), jnp.int32),
            pltpu.VMEM((EC,), jnp.int32),
            pltpu.VMEM((EC,), jnp.int32),
            pltpu.VMEM((EC,), jnp.float32),
            pltpu.VMEM((EC,), jnp.float32),
            pltpu.VMEM((L,), jnp.float32),
            pltpu.VMEM((L,), jnp.float32),
            pltpu.VMEM((L,), jnp.int32),
            pltpu.VMEM((2048,), jnp.float32),
            pltpu.VMEM((L,), jnp.int32),
            pltpu.VMEM_SHARED((CNTB,), jnp.float32),
        ],
        compiler_params=_SC_PARAMS,
        name="rgcn_prologue",
    )(_prologue_body)


HB = 8


def _edge_pair(ybuf, e0, lane0, cfbuf, dlbuf, acc):
    it2 = lax.iota(jnp.int32, L)
    ef0 = jnp.full((L,), 0, jnp.int32) + lane0
    ef1 = ef0 + 1
    base0 = plsc.load_gather(dlbuf, [ef0]) * D + it2
    base1 = plsc.load_gather(dlbuf, [ef1]) * D + it2
    cf0 = [plsc.load_gather(cfbuf, [jnp.full((L,), bb * L, jnp.int32) + ef0])
           for bb in range(NB)]
    cf1 = [plsc.load_gather(cfbuf, [jnp.full((L,), bb * L, jnp.int32) + ef1])
           for bb in range(NB)]
    for cc in range(DL):
        y0 = [ybuf[e0, pl.ds(bb * D + cc * L, L)] for bb in range(NB)]
        y1 = [ybuf[e0 + 1, pl.ds(bb * D + cc * L, L)] for bb in range(NB)]
        a0 = ((cf0[0] * y0[0] + cf0[1] * y0[1])
              + (cf0[2] * y0[2] + cf0[3] * y0[3])) \
            + ((cf0[4] * y0[4] + cf0[5] * y0[5])
               + (cf0[6] * y0[6] + cf0[7] * y0[7]))
        a1 = ((cf1[0] * y1[0] + cf1[1] * y1[1])
              + (cf1[2] * y1[2] + cf1[3] * y1[3])) \
            + ((cf1[4] * y1[4] + cf1[5] * y1[5])
               + (cf1[6] * y1[6] + cf1[7] * y1[7]))
        plsc.addupdate_scatter(acc, [base0 + cc * L], a0)
        plsc.addupdate_scatter(acc, [base1 + cc * L], a1)


def _aggregate_body(esrc_h, edl_h, ety_h, enorm_h, cnts_h, comp_h, y_h,
                    apad_o,
                    sbuf, dbuf, tbuf, nbuf, compv, cbuf, ybA, ybB, cfbuf,
                    dlbuf, idxb, acc, semA, semB):
    c = lax.axis_index("c")
    s = lax.axis_index("s")
    w = c * NS + s
    pltpu.sync_copy(comp_h, compv)

    def zz(k, _):
        acc[pl.ds(k * L, L)] = jnp.zeros((L,), jnp.float32)
        return 0
    lax.fori_loop(0, ACC_R * D // L, zz, 0)

    def src_chunk(s1, _):
        slot = (s1 * NW + w) * ECP
        pltpu.sync_copy(cnts_h.at[pl.ds((s1 * NW + w) * L, L)], cbuf)
        count = cbuf[...][0]
        nch = (count + (CHK - 1)) // CHK

        def chunk_body(ch, _):
            cbase = ch * CHK
            ck = jnp.minimum(CHK, count - cbase)
            pltpu.sync_copy(esrc_h.at[pl.ds(slot + cbase, CHK)], sbuf)
            pltpu.sync_copy(edl_h.at[pl.ds(slot + cbase, CHK)], dbuf)
            pltpu.sync_copy(ety_h.at[pl.ds(slot + cbase, CHK)], tbuf)
            pltpu.sync_copy(enorm_h.at[pl.ds(slot + cbase, CHK)], nbuf)
            nblk = (ck + (L - 1)) // L

            it0 = lax.iota(jnp.int32, L)
            idxb[...] = jnp.where(it0 < ck, sbuf[pl.ds(0, L)], 0)
            pltpu.async_copy(y_h.at[idxb.at[pl.ds(0, HB)]], ybA, semA)

            def block_body(bk, _):
                base = bk * L
                it = lax.iota(jnp.int32, L)
                valid = (base + it) < ck
                tyv = jnp.where(valid, tbuf[pl.ds(base, L)], 0)
                normv = jnp.where(valid, nbuf[pl.ds(base, L)], 0.0)
                for bb in range(NB):
                    cfbuf[pl.ds(bb * L, L)] = normv * plsc.load_gather(
                        compv, [tyv, jnp.full((L,), bb, jnp.int32)])
                dlbuf[...] = jnp.where(valid, dbuf[pl.ds(base, L)], OWN)
                nv = jnp.minimum(L, ck - base)

                pltpu.make_async_copy(
                    y_h.at[pl.ds(0, HB)], ybA, semA).wait()
                pltpu.async_copy(y_h.at[idxb.at[pl.ds(HB, HB)]], ybB, semB)

                nva = jnp.minimum(nv, HB)

                def edge_a(e, _):
                    _edge_pair(ybA, e * 2, e * 2, cfbuf, dlbuf, acc)
                    return 0
                lax.fori_loop(0, (nva + 1) // 2, edge_a, 0)

                pltpu.make_async_copy(
                    y_h.at[pl.ds(0, HB)], ybB, semB).wait()

                @pl.when(bk + 1 < nblk)
                def _():
                    base2 = base + L
                    it3 = lax.iota(jnp.int32, L)
                    v2 = (base2 + it3) < ck
                    idxb[...] = jnp.where(v2, sbuf[pl.ds(base2, L)], 0)
                    pltpu.async_copy(y_h.at[idxb.at[pl.ds(0, HB)]], ybA, semA)

                nvb = jnp.maximum(nv - HB, 0)

                def edge_b(e, _):
                    _edge_pair(ybB, e * 2, HB + e * 2, cfbuf, dlbuf, acc)
                    return 0
                lax.fori_loop(0, (nvb + 1) // 2, edge_b, 0)
                return 0
            lax.fori_loop(0, nblk, block_body, 0)
            return 0
        lax.fori_loop(0, nch, chunk_body, 0)
        return 0
    lax.fori_loop(0, NS, src_chunk, 0)

    pltpu.sync_copy(acc, apad_o.at[pl.ds(w * ACC_R * D, ACC_R * D)])


def _make_aggregate():
    return functools.partial(
        pl.kernel,
        out_type=jax.ShapeDtypeStruct((NW * ACC_R * D,), jnp.float32),
        mesh=_MESH,
        scratch_types=[
            pltpu.VMEM((CHK,), jnp.int32),
            pltpu.VMEM((CHK,), jnp.int32),
            pltpu.VMEM((CHK,), jnp.int32),
            pltpu.VMEM((CHK,), jnp.float32),
            pltpu.VMEM((R, NB), jnp.float32),
            pltpu.VMEM((L,), jnp.int32),
            pltpu.VMEM((HB, NBD), jnp.float32),
            pltpu.VMEM((HB, NBD), jnp.float32),
            pltpu.VMEM((NB * L,), jnp.float32),
            pltpu.VMEM((L,), jnp.int32),
            pltpu.VMEM((L,), jnp.int32),
            pltpu.VMEM((ACC_R * D,), jnp.float32),
            pltpu.SemaphoreType.DMA,
            pltpu.SemaphoreType.DMA,
        ],
        compiler_params=_SC_PARAMS,
        name="rgcn_aggregate",
    )(_aggregate_body)


_ROWS = 400


def _mm1_body(x_ref, w_ref, yb_ref, yr_ref):
    acc = lax.dot_general(x_ref[...], w_ref[...], (((1,), (0,)), ((), ())),
                          preferred_element_type=jnp.float32)
    yb_ref[...] = acc[:, :NBD]
    yr_ref[...] = acc[:, NBD:]


def _mm1(x, w):
    return pl.pallas_call(
        _mm1_body,
        grid=(N // _ROWS,),
        in_specs=[
            pl.BlockSpec((_ROWS, D), lambda i: (i, 0)),
            pl.BlockSpec((D, NBD + D), lambda i: (0, 0)),
        ],
        out_specs=[
            pl.BlockSpec((_ROWS, NBD), lambda i: (i, 0)),
            pl.BlockSpec((_ROWS, D), lambda i: (i, 0)),
        ],
        out_shape=[
            jax.ShapeDtypeStruct((N, NBD), jnp.float32),
            jax.ShapeDtypeStruct((N, D), jnp.float32),
        ],
    )(x, w)


def _mm2_body(a_ref, yr_ref, b_ref, w_ref, yb_ref, yr2_ref):
    x1 = jnp.maximum(a_ref[...] + yr_ref[...] + b_ref[...], 0.0)
    acc = lax.dot_general(x1, w_ref[...], (((1,), (0,)), ((), ())),
                          preferred_element_type=jnp.float32)
    yb_ref[...] = acc[:, :NBD]
    yr2_ref[...] = acc[:, NBD:]


def _mm2(a1, yr1, bias1, w2):
    return pl.pallas_call(
        _mm2_body,
        grid=(N // _ROWS,),
        in_specs=[
            pl.BlockSpec((_ROWS, D), lambda i: (i, 0)),
            pl.BlockSpec((_ROWS, D), lambda i: (i, 0)),
            pl.BlockSpec((1, D), lambda i: (0, 0)),
            pl.BlockSpec((D, NBD + D), lambda i: (0, 0)),
        ],
        out_specs=[
            pl.BlockSpec((_ROWS, NBD), lambda i: (i, 0)),
            pl.BlockSpec((_ROWS, D), lambda i: (i, 0)),
        ],
        out_shape=[
            jax.ShapeDtypeStruct((N, NBD), jnp.float32),
            jax.ShapeDtypeStruct((N, D), jnp.float32),
        ],
    )(a1, yr1, bias1, w2)


def _fin_body(a_ref, yr_ref, b_ref, o_ref):
    o_ref[...] = a_ref[...] + yr_ref[...] + b_ref[...]


def _fin(a2, yr2, bias2):
    return pl.pallas_call(
        _fin_body,
        grid=(N // _ROWS,),
        in_specs=[
            pl.BlockSpec((_ROWS, D), lambda i: (i, 0)),
            pl.BlockSpec((_ROWS, D), lambda i: (i, 0)),
            pl.BlockSpec((1, D), lambda i: (0, 0)),
        ],
        out_specs=pl.BlockSpec((_ROWS, D), lambda i: (i, 0)),
        out_shape=jax.ShapeDtypeStruct((N, D), jnp.float32),
    )(a2, yr2, bias2)


def kernel(entity_ids, edge_index, edge_type, emb, basis1, comp1, root1, bias1,
           basis2, comp2, root2, bias2):
    src = edge_index[0]
    dst = edge_index[1]
    x = emb[entity_ids]

    esrc, edl, ety, enorm, cnts = _make_prologue()(src, dst, edge_type)

    w1 = jnp.concatenate(
        [basis1.transpose(1, 0, 2).reshape(D, NBD), root1], axis=1)
    w2 = jnp.concatenate(
        [basis2.transpose(1, 0, 2).reshape(D, NBD), root2], axis=1)

    def _unpad(apad):
        return apad.reshape(NW, ACC_R, D)[:, :OWN].reshape(NW * OWN, D)[:N]

    yb1, yr1 = _mm1(x, w1)
    a1 = _unpad(_make_aggregate()(esrc, edl, ety, enorm, cnts, comp1, yb1))
    yb2, yr2 = _mm2(a1, yr1, bias1[None, :], w2)
    a2 = _unpad(_make_aggregate()(esrc, edl, ety, enorm, cnts, comp2, yb2))
    return _fin(a2, yr2, bias2[None, :])
